# baseline (speedup 1.0000x reference)
import functools

import jax
import jax.numpy as jnp
from jax import lax
from jax.experimental import pallas as pl
from jax.experimental.pallas import tpu as pltpu
from jax.experimental.pallas import tpu_sc as plsc

NUM_EMB = 1000000
D = 64
B, L = 16384, 50
N = B * L

NC, NS = 2, 16
NW = NC * NS
PER_W = N // NW
CHUNK = 128
GROUP = 8
ROWS = CHUNK * GROUP
ROUNDS = PER_W // ROWS
N_CHUNKS = PER_W // CHUNK


def _sc_gather(idx, table):
    mesh = plsc.VectorSubcoreMesh(core_axis_name="c", subcore_axis_name="s")

    @functools.partial(
        pl.kernel,
        mesh=mesh,
        out_type=jax.ShapeDtypeStruct((N, 2 * D), jnp.float32),
        compiler_params=pltpu.CompilerParams(use_tc_tiling_on_sc=False),
        scratch_types=[
            pltpu.VMEM((N_CHUNKS, CHUNK), jnp.int32),
            pltpu.VMEM((ROWS, D), jnp.float32),
            pltpu.SemaphoreType.DMA,
        ],
    )
    def k(idx_hbm, table_hbm, out_hbm, idx_v, rows_v, sem):
        wid = lax.axis_index("s") * NC + lax.axis_index("c")
        base = wid * PER_W
        pltpu.sync_copy(idx_hbm.at[wid], idx_v)

        def round_body(r, carry):
            handles = []
            for g in range(GROUP):
                h = pltpu.async_copy(
                    table_hbm.at[idx_v.at[r * GROUP + g]],
                    rows_v.at[pl.ds(g * CHUNK, CHUNK)],
                    sem,
                )
                handles.append(h)
            for h in handles:
                h.wait()
            pltpu.sync_copy(
                rows_v,
                out_hbm.at[pl.ds(base + r * ROWS, ROWS), pl.ds(0, D)],
            )
            return carry

        lax.fori_loop(0, ROUNDS, round_body, 0)

    return k(idx, table)


def _pack_body(x_ref, se_ref, so_ref, o_ref):
    for j in range(8):
        for t in range(4):
            xs = x_ref[:, j, t * 250:(t + 1) * 250]
            ye = lax.dot_general(
                se_ref[...], xs, (((1,), (1,)), ((), ())),
                preferred_element_type=jnp.float32)
            yo = lax.dot_general(
                so_ref[...], xs, (((1,), (1,)), ((), ())),
                preferred_element_type=jnp.float32)
            o_ref[pl.ds(j * 500 + t * 125, 125), :] = (
                jnp.concatenate([ye, yo], axis=1))


def _tc_pack_table(tbl3, se, so):
    return pl.pallas_call(
        _pack_body,
        grid=(125,),
        in_specs=[
            pl.BlockSpec((D, 8, 1000), lambda g: (0, g, 0)),
            pl.BlockSpec((125, 250), lambda g: (0, 0)),
            pl.BlockSpec((125, 250), lambda g: (0, 0)),
        ],
        out_specs=pl.BlockSpec((4000, 128), lambda g: (g, 0)),
        out_shape=jax.ShapeDtypeStruct((NUM_EMB // 2, 128), jnp.float32),
    )(tbl3, se, so)


BN = 2048


def _mm_body(x_ref, w_ref, o_ref):
    x = x_ref[:, 0:D]
    y = lax.dot_general(
        w_ref[...], x,
        (((1,), (1,)), ((), ())),
        preferred_element_type=jnp.float32,
    )
    o_ref[...] = y.reshape(1, D, BN)


def _tc_project(gathered_t, w):
    return pl.pallas_call(
        _mm_body,
        grid=(L, B // BN),
        in_specs=[
            pl.BlockSpec((BN, 2 * D), lambda l, i: (l * (B // BN) + i, 0)),
            pl.BlockSpec((D, D), lambda l, i: (0, 0)),
        ],
        out_specs=pl.BlockSpec((1, D, BN), lambda l, i: (l, 0, i)),
        out_shape=jax.ShapeDtypeStruct((L, D, B), jnp.float32),
    )(gathered_t, w)


def kernel(input_ids, embedding_matrix_1, embedding_matrix_2):
    idx = input_ids.T.reshape(NW, N_CHUNKS, CHUNK).astype(jnp.int32)
    tbl3 = embedding_matrix_1.T.reshape(D, 1000, 1000)
    r = jnp.arange(125)
    se = jnp.zeros((125, 250), jnp.float32).at[r, 2 * r].set(1.0)
    so = jnp.zeros((125, 250), jnp.float32).at[r, 2 * r + 1].set(1.0)
    table_lin = _tc_pack_table(tbl3, se, so).reshape(NUM_EMB, D)
    gathered_t = _sc_gather(idx, table_lin)
    out3 = _tc_project(gathered_t, embedding_matrix_2)
    return out3.transpose(2, 0, 1)


# speedup vs baseline: 1.1618x; 1.1618x over previous
import functools

import jax
import jax.numpy as jnp
from jax import lax
from jax.experimental import pallas as pl
from jax.experimental.pallas import tpu as pltpu
from jax.experimental.pallas import tpu_sc as plsc

NUM_EMB = 1000000
D = 64
B, L = 16384, 50
N = B * L

NC, NS = 2, 16
NW = NC * NS
PER_W = N // NW
CHUNK = 128
GROUP = 8
ROWS = CHUNK * GROUP
ROUNDS = PER_W // ROWS
N_CHUNKS = PER_W // CHUNK


def _sc_gather(idx, table):
    mesh = plsc.VectorSubcoreMesh(core_axis_name="c", subcore_axis_name="s")

    @functools.partial(
        pl.kernel,
        mesh=mesh,
        out_type=jax.ShapeDtypeStruct((N, 2 * D), jnp.float32),
        compiler_params=pltpu.CompilerParams(use_tc_tiling_on_sc=False),
        scratch_types=[
            pltpu.VMEM((N_CHUNKS, CHUNK), jnp.int32),
            pltpu.VMEM((ROWS, D), jnp.float32),
            pltpu.SemaphoreType.DMA,
        ],
    )
    def k(idx_hbm, table_hbm, out_hbm, idx_v, rows_v, sem):
        wid = lax.axis_index("s") * NC + lax.axis_index("c")
        base = wid * PER_W
        pltpu.sync_copy(idx_hbm.at[wid], idx_v)

        def round_body(r, carry):
            handles = []
            for g in range(GROUP):
                h = pltpu.async_copy(
                    table_hbm.at[idx_v.at[r * GROUP + g]],
                    rows_v.at[pl.ds(g * CHUNK, CHUNK)],
                    sem,
                )
                handles.append(h)
            for h in handles:
                h.wait()
            pltpu.sync_copy(
                rows_v,
                out_hbm.at[pl.ds(base + r * ROWS, ROWS), pl.ds(0, D)],
            )
            return carry

        lax.fori_loop(0, ROUNDS, round_body, 0)

    return k(idx, table)


BN = 2048


def _mm_body(x_ref, w_ref, o_ref):
    x = x_ref[:, 0:D]
    y = lax.dot_general(
        w_ref[...], x,
        (((1,), (1,)), ((), ())),
        preferred_element_type=jnp.float32,
    )
    o_ref[...] = y.reshape(1, D, BN)


def _tc_project(gathered_t, w):
    return pl.pallas_call(
        _mm_body,
        grid=(L, B // BN),
        in_specs=[
            pl.BlockSpec((BN, 2 * D), lambda l, i: (l * (B // BN) + i, 0)),
            pl.BlockSpec((D, D), lambda l, i: (0, 0)),
        ],
        out_specs=pl.BlockSpec((1, D, BN), lambda l, i: (l, 0, i)),
        out_shape=jax.ShapeDtypeStruct((L, D, B), jnp.float32),
    )(gathered_t, w)


def kernel(input_ids, embedding_matrix_1, embedding_matrix_2):
    idx = input_ids.T.reshape(NW, N_CHUNKS, CHUNK).astype(jnp.int32)
    gathered_t = _sc_gather(idx, embedding_matrix_1)
    out3 = _tc_project(gathered_t, embedding_matrix_2)
    return out3.transpose(2, 0, 1)


# speedup vs baseline: 1.2830x; 1.1043x over previous
import functools

import jax
import jax.numpy as jnp
from jax import lax
from jax.experimental import pallas as pl
from jax.experimental.pallas import tpu as pltpu
from jax.experimental.pallas import tpu_sc as plsc

NUM_EMB = 1000000
D = 64
B, L = 16384, 50
N = B * L

NC, NS = 2, 16
NW = NC * NS
PER_W = N // NW
CHUNK = 128
GROUP = 8
ROWS = CHUNK * GROUP
ROUNDS = PER_W // ROWS
N_CHUNKS = PER_W // CHUNK


def _sc_gather(idx, table):
    mesh = plsc.VectorSubcoreMesh(core_axis_name="c", subcore_axis_name="s")

    @functools.partial(
        pl.kernel,
        mesh=mesh,
        out_type=jax.ShapeDtypeStruct((N, 2 * D), jnp.float32),
        compiler_params=pltpu.CompilerParams(use_tc_tiling_on_sc=False),
        scratch_types=[
            pltpu.VMEM((N_CHUNKS, CHUNK), jnp.int32),
            pltpu.VMEM((ROWS, D), jnp.float32),
            pltpu.SemaphoreType.DMA,
        ],
    )
    def k(idx_hbm, table_hbm, out_hbm, idx_v, rows_v, sem):
        wid = lax.axis_index("s") * NC + lax.axis_index("c")
        base = wid * PER_W
        pltpu.sync_copy(idx_hbm.at[wid], idx_v)

        def round_body(r, carry):
            handles = []
            for g in range(GROUP):
                h = pltpu.async_copy(
                    table_hbm.at[idx_v.at[r * GROUP + g]],
                    rows_v.at[pl.ds(g * CHUNK, CHUNK)],
                    sem,
                )
                handles.append(h)
            for h in handles:
                h.wait()
            pltpu.sync_copy(
                rows_v,
                out_hbm.at[pl.ds(base + r * ROWS, ROWS), pl.ds(0, D)],
            )
            return carry

        lax.fori_loop(0, ROUNDS, round_body, 0)

    return k(idx, table)


BN = 4096


def _mm_body(x_ref, w_ref, o_ref):
    x = x_ref[:, 0:D]
    y = lax.dot_general(
        w_ref[...], x,
        (((1,), (1,)), ((), ())),
        preferred_element_type=jnp.float32,
    )
    o_ref[...] = y.reshape(1, D, BN)


def _tc_project(gathered_t, w):
    return pl.pallas_call(
        _mm_body,
        grid=(L, B // BN),
        in_specs=[
            pl.BlockSpec((BN, 2 * D), lambda l, i: (l * (B // BN) + i, 0)),
            pl.BlockSpec((D, D), lambda l, i: (0, 0)),
        ],
        out_specs=pl.BlockSpec((1, D, BN), lambda l, i: (l, 0, i)),
        out_shape=jax.ShapeDtypeStruct((L, D, B), jnp.float32),
    )(gathered_t, w)


def kernel(input_ids, embedding_matrix_1, embedding_matrix_2):
    idx = input_ids.T.reshape(NW, N_CHUNKS, CHUNK).astype(jnp.int32)
    gathered_t = _sc_gather(idx, embedding_matrix_1)
    out3 = _tc_project(gathered_t, embedding_matrix_2)
    return out3.transpose(2, 0, 1)


# speedup vs baseline: 1.3679x; 1.0661x over previous
import functools

import jax
import jax.numpy as jnp
from jax import lax
from jax.experimental import pallas as pl
from jax.experimental.pallas import tpu as pltpu
from jax.experimental.pallas import tpu_sc as plsc

NUM_EMB = 1000000
D = 64
B, L = 16384, 50
N = B * L

NC, NS = 2, 16
NW = NC * NS
PER_W = N // NW
CHUNK = 128
GROUP = 8
ROWS = CHUNK * GROUP
ROUNDS = PER_W // ROWS
N_CHUNKS = PER_W // CHUNK


def _sc_gather(idx, table):
    mesh = plsc.VectorSubcoreMesh(core_axis_name="c", subcore_axis_name="s")

    @functools.partial(
        pl.kernel,
        mesh=mesh,
        out_type=jax.ShapeDtypeStruct((N, 2 * D), jnp.float32),
        compiler_params=pltpu.CompilerParams(use_tc_tiling_on_sc=False),
        scratch_types=[
            pltpu.VMEM((N_CHUNKS, CHUNK), jnp.int32),
            pltpu.VMEM((ROWS, D), jnp.float32),
            pltpu.SemaphoreType.DMA,
        ],
    )
    def k(idx_hbm, table_hbm, out_hbm, idx_v, rows_v, sem):
        wid = lax.axis_index("s") * NC + lax.axis_index("c")
        base = wid * PER_W
        pltpu.sync_copy(idx_hbm.at[wid], idx_v)

        def round_body(r, carry):
            handles = []
            for g in range(GROUP):
                h = pltpu.async_copy(
                    table_hbm.at[idx_v.at[r * GROUP + g]],
                    rows_v.at[pl.ds(g * CHUNK, CHUNK)],
                    sem,
                )
                handles.append(h)
            for h in handles:
                h.wait()
            pltpu.sync_copy(
                rows_v,
                out_hbm.at[pl.ds(base + r * ROWS, ROWS), pl.ds(0, D)],
            )
            return carry

        lax.fori_loop(0, ROUNDS, round_body, 0)

    return k(idx, table)


BN = 8192


def _mm_body(x_ref, w_ref, o_ref):
    x = x_ref[:, 0:D]
    y = lax.dot_general(
        w_ref[...], x,
        (((1,), (1,)), ((), ())),
        preferred_element_type=jnp.float32,
    )
    o_ref[...] = y.reshape(1, D, BN)


def _tc_project(gathered_t, w):
    return pl.pallas_call(
        _mm_body,
        grid=(L, B // BN),
        in_specs=[
            pl.BlockSpec((BN, 2 * D), lambda l, i: (l * (B // BN) + i, 0)),
            pl.BlockSpec((D, D), lambda l, i: (0, 0)),
        ],
        out_specs=pl.BlockSpec((1, D, BN), lambda l, i: (l, 0, i)),
        out_shape=jax.ShapeDtypeStruct((L, D, B), jnp.float32),
    )(gathered_t, w)


def kernel(input_ids, embedding_matrix_1, embedding_matrix_2):
    idx = input_ids.T.reshape(NW, N_CHUNKS, CHUNK).astype(jnp.int32)
    gathered_t = _sc_gather(idx, embedding_matrix_1)
    out3 = _tc_project(gathered_t, embedding_matrix_2)
    return out3.transpose(2, 0, 1)


# speedup vs baseline: 1.3802x; 1.0090x over previous
import functools

import jax
import jax.numpy as jnp
from jax import lax
from jax.experimental import pallas as pl
from jax.experimental.pallas import tpu as pltpu
from jax.experimental.pallas import tpu_sc as plsc

NUM_EMB = 1000000
D = 64
B, L = 16384, 50
N = B * L

NC, NS = 2, 16
NW = NC * NS
PER_W = N // NW
CHUNK = 128
GROUP = 8
ROWS = CHUNK * GROUP
ROUNDS = PER_W // ROWS
N_CHUNKS = PER_W // CHUNK


def _sc_gather(idx, table):
    mesh = plsc.VectorSubcoreMesh(core_axis_name="c", subcore_axis_name="s")

    @functools.partial(
        pl.kernel,
        mesh=mesh,
        out_type=jax.ShapeDtypeStruct((N, 2 * D), jnp.float32),
        compiler_params=pltpu.CompilerParams(use_tc_tiling_on_sc=False),
        scratch_types=[
            pltpu.VMEM((N_CHUNKS, CHUNK), jnp.int32),
            pltpu.VMEM((ROWS, D), jnp.float32),
            pltpu.SemaphoreType.DMA,
        ],
    )
    def k(idx_hbm, table_hbm, out_hbm, idx_v, rows_v, sem):
        wid = lax.axis_index("s") * NC + lax.axis_index("c")
        base = wid * PER_W
        pltpu.sync_copy(idx_hbm.at[wid], idx_v)

        def round_body(r, carry):
            handles = []
            for g in range(GROUP):
                h = pltpu.async_copy(
                    table_hbm.at[idx_v.at[r * GROUP + g]],
                    rows_v.at[pl.ds(g * CHUNK, CHUNK)],
                    sem,
                )
                handles.append(h)
            for h in handles:
                h.wait()
            pltpu.sync_copy(
                rows_v,
                out_hbm.at[pl.ds(base + r * ROWS, ROWS), pl.ds(0, D)],
            )
            return carry

        lax.fori_loop(0, ROUNDS, round_body, 0)

    return k(idx, table)


BN = 16384


def _mm_body(x_ref, w_ref, o_ref):
    x = x_ref[:, 0:D]
    y = lax.dot_general(
        w_ref[...], x,
        (((1,), (1,)), ((), ())),
        preferred_element_type=jnp.float32,
    )
    o_ref[...] = y.reshape(1, D, BN)


def _tc_project(gathered_t, w):
    return pl.pallas_call(
        _mm_body,
        grid=(L, B // BN),
        in_specs=[
            pl.BlockSpec((BN, 2 * D), lambda l, i: (l * (B // BN) + i, 0)),
            pl.BlockSpec((D, D), lambda l, i: (0, 0)),
        ],
        out_specs=pl.BlockSpec((1, D, BN), lambda l, i: (l, 0, i)),
        out_shape=jax.ShapeDtypeStruct((L, D, B), jnp.float32),
    )(gathered_t, w)


def kernel(input_ids, embedding_matrix_1, embedding_matrix_2):
    idx = input_ids.T.reshape(NW, N_CHUNKS, CHUNK).astype(jnp.int32)
    gathered_t = _sc_gather(idx, embedding_matrix_1)
    out3 = _tc_project(gathered_t, embedding_matrix_2)
    return out3.transpose(2, 0, 1)


# speedup vs baseline: 1.3914x; 1.0081x over previous
import functools

import jax
import jax.numpy as jnp
from jax import lax
from jax.experimental import pallas as pl
from jax.experimental.pallas import tpu as pltpu
from jax.experimental.pallas import tpu_sc as plsc

NUM_EMB = 1000000
D = 64
B, L = 16384, 50
N = B * L

NC, NS = 2, 16
NW = NC * NS
PER_W = N // NW
CHUNK = 128
GROUP = 4
ROWS = CHUNK * GROUP
ROUNDS = PER_W // ROWS
N_CHUNKS = PER_W // CHUNK


def _sc_gather(idx, table):
    mesh = plsc.VectorSubcoreMesh(core_axis_name="c", subcore_axis_name="s")

    @functools.partial(
        pl.kernel,
        mesh=mesh,
        out_type=jax.ShapeDtypeStruct((N, 2 * D), jnp.float32),
        compiler_params=pltpu.CompilerParams(use_tc_tiling_on_sc=False),
        scratch_types=[
            pltpu.VMEM((N_CHUNKS, CHUNK), jnp.int32),
            pltpu.VMEM((2, ROWS, D), jnp.float32),
            pltpu.SemaphoreType.DMA,
            pltpu.SemaphoreType.DMA,
        ],
    )
    def k(idx_hbm, table_hbm, out_hbm, idx_v, rows_v, sem, wsem):
        wid = lax.axis_index("s") * NC + lax.axis_index("c")
        base = wid * PER_W
        pltpu.sync_copy(idx_hbm.at[wid], idx_v)

        def drain_one_write():
            pltpu.make_async_copy(
                out_hbm.at[pl.ds(0, ROWS), pl.ds(0, D)],
                rows_v.at[0],
                wsem,
            ).wait()

        def round_body(r, carry):
            buf = rows_v.at[r % 2]
            @pl.when(r >= 2)
            def _():
                drain_one_write()

            handles = []
            for g in range(GROUP):
                h = pltpu.async_copy(
                    table_hbm.at[idx_v.at[r * GROUP + g]],
                    buf.at[pl.ds(g * CHUNK, CHUNK)],
                    sem,
                )
                handles.append(h)
            for h in handles:
                h.wait()
            pltpu.async_copy(
                buf,
                out_hbm.at[pl.ds(base + r * ROWS, ROWS), pl.ds(0, D)],
                wsem,
            )
            return carry

        lax.fori_loop(0, ROUNDS, round_body, 0)
        drain_one_write()
        drain_one_write()

    return k(idx, table)


BN = 16384


def _mm_body(x_ref, w_ref, o_ref):
    x = x_ref[:, 0:D]
    y = lax.dot_general(
        w_ref[...], x,
        (((1,), (1,)), ((), ())),
        preferred_element_type=jnp.float32,
    )
    o_ref[...] = y.reshape(1, D, BN)


def _tc_project(gathered_t, w):
    return pl.pallas_call(
        _mm_body,
        grid=(L, B // BN),
        in_specs=[
            pl.BlockSpec((BN, 2 * D), lambda l, i: (l * (B // BN) + i, 0)),
            pl.BlockSpec((D, D), lambda l, i: (0, 0)),
        ],
        out_specs=pl.BlockSpec((1, D, BN), lambda l, i: (l, 0, i)),
        out_shape=jax.ShapeDtypeStruct((L, D, B), jnp.float32),
    )(gathered_t, w)


def kernel(input_ids, embedding_matrix_1, embedding_matrix_2):
    idx = input_ids.T.reshape(NW, N_CHUNKS, CHUNK).astype(jnp.int32)
    gathered_t = _sc_gather(idx, embedding_matrix_1)
    out3 = _tc_project(gathered_t, embedding_matrix_2)
    return out3.transpose(2, 0, 1)
